# baseline (speedup 1.0000x reference)
import functools

import jax
import jax.numpy as jnp
from jax import lax
from jax.experimental import pallas as pl
from jax.experimental.pallas import tpu as pltpu
from jax.experimental.pallas import tpu_sc as plsc

N = 10000
E = 320000
F = 128
ALPHA = 0.2
G = 80
EB = E // G
GPB = EB // 16
BG = 25
BGE = BG * G
NBLK = GPB // BG
FH = F // 2
N_PAD = 10240
NPT = N_PAD // 16
NLD = N // 16
NRING = 3

_sc_mesh = plsc.VectorSubcoreMesh(core_axis_name="c", subcore_axis_name="s")
_sc_params = pltpu.CompilerParams(needs_layout_passes=False,
                                  use_tc_tiling_on_sc=False)


def _proj_body(x_ref, wb_ref, w_ref, a_ref, h2_ref, ha_ref):
    h2_ref[...] = jnp.dot(x_ref[...], wb_ref[0],
                          preferred_element_type=jnp.float32)
    wa = jnp.dot(w_ref[...], a_ref[...], preferred_element_type=jnp.float32)
    ha_ref[...] = jnp.dot(x_ref[...], wa, preferred_element_type=jnp.float32)


def _project(x, W, attn2):
    blk = 1000
    nb = N // blk
    return pl.pallas_call(
        _proj_body,
        grid=(nb, 2),
        in_specs=[
            pl.BlockSpec((blk, F), lambda i, j: (i, 0)),
            pl.BlockSpec((1, F, FH), lambda i, j: (j, 0, 0)),
            pl.BlockSpec((F, F), lambda i, j: (0, 0)),
            pl.BlockSpec((F, 2), lambda i, j: (0, 0)),
        ],
        out_specs=[
            pl.BlockSpec((blk, FH), lambda i, j: (j * nb + i, 0)),
            pl.BlockSpec((blk, 2), lambda i, j: (i, 0)),
        ],
        out_shape=[
            jax.ShapeDtypeStruct((2 * N, FH), jnp.float32),
            jax.ShapeDtypeStruct((N, 2), jnp.float32),
        ],
    )(x, W.reshape(F, 2, FH).transpose(1, 0, 2), W, attn2)


@functools.partial(
    pl.kernel,
    out_type=jax.ShapeDtypeStruct((N, F), jnp.float32),
    mesh=_sc_mesh,
    scratch_types=[
        pltpu.VMEM((N,), jnp.float32),
        pltpu.VMEM((N,), jnp.float32),
        pltpu.VMEM((BGE,), jnp.int32),
        pltpu.VMEM((BGE,), jnp.int32),
        pltpu.VMEM((BGE,), jnp.float32),
        pltpu.VMEM((NRING, G, FH), jnp.float32),
        pltpu.VMEM((NPT,), jnp.float32),
        pltpu.VMEM_SHARED((N, FH), jnp.float32),
        pltpu.VMEM_SHARED((N_PAD, FH), jnp.float32),
        pltpu.VMEM_SHARED((N_PAD,), jnp.float32),
        pltpu.SemaphoreType.DMA,
        pltpu.SemaphoreType.DMA,
        pltpu.SemaphoreType.DMA,
        pltpu.SemaphoreType.DMA,
        pltpu.SemaphoreType.DMA,
        pltpu.SemaphoreType.DMA,
        pltpu.SemaphoreType.DMA,
    ],
    compiler_params=_sc_params,
)
def _gat_kernel(asrc_hbm, adst_hbm, h2_hbm, src_hbm, dst_hbm, out_hbm,
                asrc_v, adst_v, src_v, dst_v, w_v, ring_v, sbuf,
                tab_sh, acc_sh, ssum_sh,
                gs0, gs1, gs2, ss0, ss1, ss2, bsem):
    c = lax.axis_index("c")
    s = lax.axis_index("s")
    gsems = (gs0, gs1, gs2)
    ssems = (ss0, ss1, ss2)
    pltpu.sync_copy(asrc_hbm, asrc_v)
    pltpu.sync_copy(adst_hbm, adst_v)
    pltpu.sync_copy(h2_hbm.at[pl.ds(c * N + s * NLD, NLD)],
                    tab_sh.at[pl.ds(s * NLD, NLD)])

    def _zrow(e, _):
        def _zc(k, _):
            ring_v[0, e, pl.ds(k * 16, 16)] = jnp.zeros((16,), jnp.float32)
            return 0
        lax.fori_loop(0, FH // 16, _zc, 0)
        return 0
    lax.fori_loop(0, G, _zrow, 0)

    def _zs(i, _):
        sbuf[pl.ds(i * 16, 16)] = jnp.zeros((16,), jnp.float32)
        return 0
    lax.fori_loop(0, NPT // 16, _zs, 0)
    pltpu.sync_copy(sbuf, ssum_sh.at[pl.ds(s * NPT, NPT)])

    def _zout(j, _):
        pltpu.sync_copy(ring_v.at[0], acc_sh.at[pl.ds(s * NPT + j * G, G)])
        return 0
    lax.fori_loop(0, NPT // G, _zout, 0)
    plsc.subcore_barrier()

    def _mult(buf, base):
        @plsc.parallel_loop(0, G, step=16, unroll=2)
        def _e16(m):
            w16 = w_v[pl.ds(base + m, 16)]
            for e in range(16):
                wb = w16.at[jnp.full((16,), e, jnp.int32)].get(
                    mode="promise_in_bounds")
                for k in range(FH // 16):
                    ring_v[buf, m + e, pl.ds(k * 16, 16)] = (
                        ring_v[buf, m + e, pl.ds(k * 16, 16)] * wb)

    def _block(b, _):
        pltpu.sync_copy(src_hbm.at[s, b], src_v)
        pltpu.sync_copy(dst_hbm.at[s, b], dst_v)

        @plsc.parallel_loop(0, BGE, step=16, unroll=8)
        def _ev(i):
            si = src_v[pl.ds(i, 16)]
            di = dst_v[pl.ds(i, 16)]
            v = (plsc.load_gather(asrc_v, [si])
                 + plsc.load_gather(adst_v, [di]))
            w_v[pl.ds(i, 16)] = jnp.exp(
                jnp.where(v > 0, -v, (-ALPHA) * v))

        hsum = [pltpu.async_copy(w_v.at[pl.ds(g * G, G)],
                                 ssum_sh.at[src_v.at[pl.ds(g * G, G)]],
                                 bsem, add=True)
                for g in range(BG)]

        gh = [None] * NRING
        sh = [None] * NRING
        for g in range(BG + 1):
            if g < BG:
                i = g % NRING
                if sh[i] is not None:
                    sh[i].wait()
                gh[i] = pltpu.async_copy(
                    tab_sh.at[dst_v.at[pl.ds(g * G, G)]],
                    ring_v.at[i], gsems[i])
            if g >= 1:
                j = (g - 1) % NRING
                gh[j].wait()
                _mult(j, (g - 1) * G)
                sh[j] = pltpu.async_copy(
                    ring_v.at[j],
                    acc_sh.at[src_v.at[pl.ds((g - 1) * G, G)]],
                    ssems[j], add=True)
        for h in sh:
            if h is not None:
                h.wait()
        for h in hsum:
            h.wait()
        return 0
    lax.fori_loop(0, NBLK, _block, 0)
    plsc.subcore_barrier()

    pltpu.sync_copy(ssum_sh.at[pl.ds(s * NPT, NPT)], sbuf)

    def _wout(j, _):
        @pl.when(s * NPT + j * G + G <= N)
        def _valid():
            pltpu.sync_copy(acc_sh.at[pl.ds(s * NPT + j * G, G)],
                            ring_v.at[0])

            @plsc.parallel_loop(0, G, step=1, unroll=8)
            def _nrow(e):
                ib = jnp.full((16,), j * G + e, jnp.int32)
                sv = plsc.load_gather(sbuf, [ib])
                recip = 1.0 / (sv + 1e-16)
                for k in range(FH // 16):
                    val = ring_v[0, e, pl.ds(k * 16, 16)] * recip
                    ring_v[0, e, pl.ds(k * 16, 16)] = jnp.where(
                        val > 0, val, jnp.exp(jnp.minimum(val, 0.0)) - 1.0)
            pltpu.sync_copy(ring_v.at[0],
                            out_hbm.at[pl.ds(s * NPT + j * G, G),
                                       pl.ds(c * FH, FH)])
        return 0
    lax.fori_loop(0, NPT // G, _wout, 0)


def kernel(x, edge, W, attn):
    src = edge[0].astype(jnp.int32)
    dst = edge[1].astype(jnp.int32)
    attn2 = jnp.stack([attn[:F], attn[F:]], axis=1).astype(jnp.float32)
    h2, ha = _project(x.astype(jnp.float32), W.astype(jnp.float32), attn2)
    return _gat_kernel(ha[:, 0], ha[:, 1], h2,
                       src.reshape(16, NBLK, BGE), dst.reshape(16, NBLK, BGE))


# speedup vs baseline: 1.0114x; 1.0114x over previous
import functools

import jax
import jax.numpy as jnp
from jax import lax
from jax.experimental import pallas as pl
from jax.experimental.pallas import tpu as pltpu
from jax.experimental.pallas import tpu_sc as plsc

N = 10000
E = 320000
F = 128
ALPHA = 0.2
G = 80
EB = E // G
GPB = EB // 16
BG = 25
BGE = BG * G
NBLK = GPB // BG
FH = F // 2
N_PAD = 10240
NPT = N_PAD // 16
NLD = N // 16
NRING = 3

_sc_mesh = plsc.VectorSubcoreMesh(core_axis_name="c", subcore_axis_name="s")
_sc_params = pltpu.CompilerParams(needs_layout_passes=False,
                                  use_tc_tiling_on_sc=False)


def _proj_body(x_ref, wb_ref, w_ref, a_ref, h2_ref, ha_ref):
    h2_ref[...] = jnp.dot(x_ref[...], wb_ref[0],
                          preferred_element_type=jnp.float32)
    wa = jnp.dot(w_ref[...], a_ref[...], preferred_element_type=jnp.float32)
    ha_ref[...] = jnp.dot(x_ref[...], wa, preferred_element_type=jnp.float32)


def _project(x, W, attn_pad):
    blk = 1000
    nb = N // blk
    return pl.pallas_call(
        _proj_body,
        grid=(nb, 2),
        in_specs=[
            pl.BlockSpec((blk, F), lambda i, j: (i, 0)),
            pl.BlockSpec((1, F, FH), lambda i, j: (j, 0, 0)),
            pl.BlockSpec((F, F), lambda i, j: (0, 0)),
            pl.BlockSpec((F, F), lambda i, j: (0, 0)),
        ],
        out_specs=[
            pl.BlockSpec((blk, FH), lambda i, j: (j * nb + i, 0)),
            pl.BlockSpec((blk, F), lambda i, j: (i, 0)),
        ],
        out_shape=[
            jax.ShapeDtypeStruct((2 * N, FH), jnp.float32),
            jax.ShapeDtypeStruct((N, F), jnp.float32),
        ],
    )(x, W.reshape(F, 2, FH).transpose(1, 0, 2), W, attn_pad)


@functools.partial(
    pl.kernel,
    out_type=jax.ShapeDtypeStruct((N, F), jnp.float32),
    mesh=_sc_mesh,
    scratch_types=[
        pltpu.VMEM((N,), jnp.float32),
        pltpu.VMEM((N,), jnp.float32),
        pltpu.VMEM((BGE,), jnp.int32),
        pltpu.VMEM((BGE,), jnp.int32),
        pltpu.VMEM((BGE,), jnp.float32),
        pltpu.VMEM((NRING, G, FH), jnp.float32),
        pltpu.VMEM((NPT,), jnp.float32),
        pltpu.VMEM_SHARED((N, FH), jnp.float32),
        pltpu.VMEM_SHARED((N_PAD, FH), jnp.float32),
        pltpu.VMEM_SHARED((N_PAD,), jnp.float32),
        pltpu.SemaphoreType.DMA,
        pltpu.SemaphoreType.DMA,
        pltpu.SemaphoreType.DMA,
        pltpu.SemaphoreType.DMA,
        pltpu.SemaphoreType.DMA,
        pltpu.SemaphoreType.DMA,
        pltpu.SemaphoreType.DMA,
    ],
    compiler_params=_sc_params,
)
def _gat_kernel(asrc_hbm, adst_hbm, h2_hbm, src_hbm, dst_hbm, out_hbm,
                asrc_v, adst_v, src_v, dst_v, w_v, ring_v, sbuf,
                tab_sh, acc_sh, ssum_sh,
                gs0, gs1, gs2, ss0, ss1, ss2, bsem):
    c = lax.axis_index("c")
    s = lax.axis_index("s")
    gsems = (gs0, gs1, gs2)
    ssems = (ss0, ss1, ss2)
    pltpu.sync_copy(asrc_hbm, asrc_v)
    pltpu.sync_copy(adst_hbm, adst_v)
    pltpu.sync_copy(h2_hbm.at[pl.ds(c * N + s * NLD, NLD)],
                    tab_sh.at[pl.ds(s * NLD, NLD)])

    def _zrow(e, _):
        def _zc(k, _):
            ring_v[0, e, pl.ds(k * 16, 16)] = jnp.zeros((16,), jnp.float32)
            return 0
        lax.fori_loop(0, FH // 16, _zc, 0)
        return 0
    lax.fori_loop(0, G, _zrow, 0)

    def _zs(i, _):
        sbuf[pl.ds(i * 16, 16)] = jnp.zeros((16,), jnp.float32)
        return 0
    lax.fori_loop(0, NPT // 16, _zs, 0)
    pltpu.sync_copy(sbuf, ssum_sh.at[pl.ds(s * NPT, NPT)])

    def _zout(j, _):
        pltpu.sync_copy(ring_v.at[0], acc_sh.at[pl.ds(s * NPT + j * G, G)])
        return 0
    lax.fori_loop(0, NPT // G, _zout, 0)
    plsc.subcore_barrier()

    def _mult(buf, base):
        @plsc.parallel_loop(0, G, step=16, unroll=2)
        def _e16(m):
            w16 = w_v[pl.ds(base + m, 16)]
            for e in range(16):
                wb = w16.at[jnp.full((16,), e, jnp.int32)].get(
                    mode="promise_in_bounds")
                for k in range(FH // 16):
                    ring_v[buf, m + e, pl.ds(k * 16, 16)] = (
                        ring_v[buf, m + e, pl.ds(k * 16, 16)] * wb)

    def _block(b, _):
        pltpu.sync_copy(src_hbm.at[s, b], src_v)
        pltpu.sync_copy(dst_hbm.at[s, b], dst_v)

        @plsc.parallel_loop(0, BGE, step=16, unroll=4)
        def _ev(i):
            si = src_v[pl.ds(i, 16)]
            di = dst_v[pl.ds(i, 16)]
            v = (plsc.load_gather(asrc_v, [si])
                 + plsc.load_gather(adst_v, [di]))
            w_v[pl.ds(i, 16)] = jnp.exp(
                jnp.where(v > 0, -v, (-ALPHA) * v))

        hsum = [pltpu.async_copy(w_v.at[pl.ds(g * G, G)],
                                 ssum_sh.at[src_v.at[pl.ds(g * G, G)]],
                                 bsem, add=True)
                for g in range(BG)]

        gh = [None] * NRING
        sh = [None] * NRING
        for g in range(BG + 1):
            if g < BG:
                i = g % NRING
                if sh[i] is not None:
                    sh[i].wait()
                gh[i] = pltpu.async_copy(
                    tab_sh.at[dst_v.at[pl.ds(g * G, G)]],
                    ring_v.at[i], gsems[i])
            if g >= 1:
                j = (g - 1) % NRING
                gh[j].wait()
                _mult(j, (g - 1) * G)
                sh[j] = pltpu.async_copy(
                    ring_v.at[j],
                    acc_sh.at[src_v.at[pl.ds((g - 1) * G, G)]],
                    ssems[j], add=True)
        for h in sh:
            if h is not None:
                h.wait()
        for h in hsum:
            h.wait()
        return 0
    lax.fori_loop(0, NBLK, _block, 0)
    plsc.subcore_barrier()

    pltpu.sync_copy(ssum_sh.at[pl.ds(s * NPT, NPT)], sbuf)

    def _wout(j, _):
        @pl.when(s * NPT + j * G + G <= N)
        def _valid():
            pltpu.sync_copy(acc_sh.at[pl.ds(s * NPT + j * G, G)],
                            ring_v.at[0])

            @plsc.parallel_loop(0, G, step=1, unroll=4)
            def _nrow(e):
                ib = jnp.full((16,), j * G + e, jnp.int32)
                sv = plsc.load_gather(sbuf, [ib])
                recip = 1.0 / (sv + 1e-16)
                for k in range(FH // 16):
                    val = ring_v[0, e, pl.ds(k * 16, 16)] * recip
                    ring_v[0, e, pl.ds(k * 16, 16)] = jnp.where(
                        val > 0, val, jnp.exp(jnp.minimum(val, 0.0)) - 1.0)
            pltpu.sync_copy(ring_v.at[0],
                            out_hbm.at[pl.ds(s * NPT + j * G, G),
                                       pl.ds(c * FH, FH)])
        return 0
    lax.fori_loop(0, NPT // G, _wout, 0)


def kernel(x, edge, W, attn):
    src = edge[0].astype(jnp.int32)
    dst = edge[1].astype(jnp.int32)
    attn_pad = jnp.concatenate(
        [attn[:F, None], attn[F:, None], jnp.zeros((F, F - 2), jnp.float32)],
        axis=1)
    h2, ha = _project(x.astype(jnp.float32), W.astype(jnp.float32), attn_pad)
    return _gat_kernel(ha[:, 0], ha[:, 1], h2,
                       src.reshape(16, NBLK, BGE), dst.reshape(16, NBLK, BGE))
